# baseline (speedup 1.0000x reference)
import functools

import jax
import jax.numpy as jnp
from jax.experimental import pallas as pl
from jax.experimental.pallas import tpu as pltpu

_BLK = 1024


def _fused_kernel(n_blocks, n_task, n_token, d_inner, d_att,
                  x_ref, wd_ref, bd_ref, wv_ref, bv_ref, wu_ref, bu_ref,
                  wa_ref, ba_ref,
                  wc0_ref, wc1_ref, bcr_ref, ws0_ref, ws1_ref, bs_ref, s_ref,
                  aout_ref, outs_ref, bags_ref,
                  h_ref, acc_ref, l_ref):
    nb = pl.program_id(0)

    @pl.when(nb == 0)
    def _init():
        h_ref[...] = jnp.zeros_like(h_ref)
        acc_ref[...] = jnp.zeros_like(acc_ref)
        l_ref[...] = jnp.zeros_like(l_ref)

    def consume(col, weight):
        h = h_ref[...]
        for i in range(n_task):
            wv_i = wv_ref[i * d_inner:(i + 1) * d_inner, :]
            wu_i = wu_ref[i * d_inner:(i + 1) * d_inner, :]
            gv = jnp.tanh(
                jnp.dot(h, wv_i, preferred_element_type=jnp.float32)
                + bv_ref[0:1, i * d_att:(i + 1) * d_att])
            gu = jax.nn.sigmoid(
                jnp.dot(h, wu_i, preferred_element_type=jnp.float32)
                + bu_ref[0:1, i * d_att:(i + 1) * d_att])
            gate_i = gv * gu
            wa_i = wa_ref[i * d_att:(i + 1) * d_att, :]
            a_i = jax.lax.dot_general(
                wa_i, gate_i, (((0,), (1,)), ((), ())),
                preferred_element_type=jnp.float32) \
                + ba_ref[i * n_token:(i + 1) * n_token, :]
            aout_ref[i * n_token:(i + 1) * n_token, pl.ds(col, _BLK)] = a_i
        a_blk = aout_ref[:, pl.ds(col, _BLK)]
        p = jnp.exp(a_blk) * weight
        l_ref[...] = l_ref[...] + jnp.sum(p, axis=1, keepdims=True)
        acc_ref[...] = acc_ref[...] + jnp.dot(
            p, h, preferred_element_type=jnp.float32)

    prev_col = jnp.maximum(nb - 1, 0) * _BLK
    consume(prev_col, jnp.where(nb > 0, 1.0, 0.0))

    xb = x_ref[...]
    h_ref[...] = jnp.maximum(
        jnp.dot(xb, wd_ref[...], preferred_element_type=jnp.float32)
        + bd_ref[...], 0.0)

    @pl.when(nb == n_blocks - 1)
    def _finish():
        consume((n_blocks - 1) * _BLK, 1.0)
        afeat = acc_ref[...] / l_ref[...]
        o0 = jnp.sum(afeat * wc0_ref[...], axis=1, keepdims=True)
        o1 = jnp.sum(afeat * wc1_ref[...], axis=1, keepdims=True)
        outs_ref[...] = jnp.concatenate([o0, o1], axis=1) + bcr_ref[...]
        bag = jnp.dot(s_ref[...], afeat,
                      preferred_element_type=jnp.float32)
        b0 = jnp.sum(bag * ws0_ref[...], axis=1, keepdims=True)
        b1 = jnp.sum(bag * ws1_ref[...], axis=1, keepdims=True)
        bags_ref[...] = jnp.concatenate([b0, b1], axis=1) + bs_ref[...]


@jax.jit
def kernel(x, Wd, bd, Wv, bv, Wu, bu, Wa, ba, Wc, bc, Ws, bs):
    n = x.shape[1]
    d_feat = x.shape[2]
    d_inner = Wd.shape[1]
    n_task, _, d_att = Wv.shape
    n_token = Wa.shape[2]
    n_class = Wc.shape[3]
    r = n_task * n_token
    n_blocks = n // _BLK

    x2 = x[0]
    Wv2 = Wv.reshape(n_task * d_inner, d_att)
    Wu2 = Wu.reshape(n_task * d_inner, d_att)
    Wa2 = Wa.reshape(n_task * d_att, n_token)
    bv2 = bv.reshape(1, n_task * d_att)
    bu2 = bu.reshape(1, n_task * d_att)
    ba_col = ba.reshape(r, 1)
    Wc_r = Wc.reshape(r, d_inner, n_class)
    bc_r = bc.reshape(r, n_class)
    S = jnp.repeat(jnp.eye(n_task, dtype=x.dtype), n_token, axis=1) / n_token

    body = functools.partial(_fused_kernel, n_blocks, n_task, n_token,
                             d_inner, d_att)
    aout, outs, bags = pl.pallas_call(
        body,
        grid=(n_blocks,),
        in_specs=[
            pl.BlockSpec((_BLK, d_feat), lambda nb: (nb, 0)),
            pl.BlockSpec((d_feat, d_inner), lambda nb: (0, 0)),
            pl.BlockSpec((1, d_inner), lambda nb: (0, 0)),
            pl.BlockSpec((n_task * d_inner, d_att), lambda nb: (0, 0)),
            pl.BlockSpec((1, n_task * d_att), lambda nb: (0, 0)),
            pl.BlockSpec((n_task * d_inner, d_att), lambda nb: (0, 0)),
            pl.BlockSpec((1, n_task * d_att), lambda nb: (0, 0)),
            pl.BlockSpec((n_task * d_att, n_token), lambda nb: (0, 0)),
            pl.BlockSpec((r, 1), lambda nb: (0, 0)),
            pl.BlockSpec((r, d_inner), lambda nb: (0, 0)),
            pl.BlockSpec((r, d_inner), lambda nb: (0, 0)),
            pl.BlockSpec((r, n_class), lambda nb: (0, 0)),
            pl.BlockSpec((n_task, d_inner), lambda nb: (0, 0)),
            pl.BlockSpec((n_task, d_inner), lambda nb: (0, 0)),
            pl.BlockSpec((n_task, n_class), lambda nb: (0, 0)),
            pl.BlockSpec((n_task, r), lambda nb: (0, 0)),
        ],
        out_specs=[
            pl.BlockSpec((r, n), lambda nb: (0, 0)),
            pl.BlockSpec((r, n_class), lambda nb: (0, 0)),
            pl.BlockSpec((n_task, n_class), lambda nb: (0, 0)),
        ],
        out_shape=[
            jax.ShapeDtypeStruct((r, n), jnp.float32),
            jax.ShapeDtypeStruct((r, n_class), jnp.float32),
            jax.ShapeDtypeStruct((n_task, n_class), jnp.float32),
        ],
        scratch_shapes=[
            pltpu.VMEM((_BLK, d_inner), jnp.float32),
            pltpu.VMEM((r, d_inner), jnp.float32),
            pltpu.VMEM((r, 1), jnp.float32),
        ],
    )(x2, Wd, bd[None, :], Wv2, bv2, Wu2, bu2, Wa2, ba_col,
      Wc_r[:, :, 0], Wc_r[:, :, 1], bc_r, Ws[:, :, 0], Ws[:, :, 1], bs, S)

    outs_full = outs.reshape(n_task, n_token, n_class)
    bags_full = bags.reshape(n_task, 1, n_class)
    aouts_full = aout.reshape(n_task, n_token, n)[:, None, :, :]
    return outs_full, bags_full, aouts_full


# speedup vs baseline: 1.3317x; 1.3317x over previous
import functools

import jax
import jax.numpy as jnp
from jax.experimental import pallas as pl
from jax.experimental.pallas import tpu as pltpu

_BLK = 1024


def _fused_kernel(n_blocks, gate_cols,
                  x_ref, wd_ref, bd_ref, wvu_ref, bvu_ref, wat_ref, ba_ref,
                  wc0_ref, wc1_ref, bcr_ref, ws0_ref, ws1_ref, bs_ref, s_ref,
                  aout_ref, outs_ref, bags_ref,
                  h_ref, acc_ref, l_ref):
    nb = pl.program_id(0)

    @pl.when(nb == 0)
    def _init():
        h_ref[...] = jnp.zeros_like(h_ref)
        acc_ref[...] = jnp.zeros_like(acc_ref)
        l_ref[...] = jnp.zeros_like(l_ref)

    def consume(col, weight):
        h = h_ref[...]
        g = jnp.dot(h, wvu_ref[...], preferred_element_type=jnp.float32) \
            + bvu_ref[...]
        gate = jnp.tanh(g[:, :gate_cols]) * jax.nn.sigmoid(g[:, gate_cols:])
        a_t = jax.lax.dot_general(
            wat_ref[...], gate, (((0,), (1,)), ((), ())),
            preferred_element_type=jnp.float32) + ba_ref[...]
        aout_ref[:, pl.ds(col, _BLK)] = a_t
        p = jnp.exp(a_t) * weight
        l_ref[...] = l_ref[...] + jnp.sum(p, axis=1, keepdims=True)
        acc_ref[...] = acc_ref[...] + jnp.dot(
            p, h, preferred_element_type=jnp.float32)

    prev_col = jnp.maximum(nb - 1, 0) * _BLK
    consume(prev_col, jnp.where(nb > 0, 1.0, 0.0))

    xb = x_ref[...]
    h_ref[...] = jnp.maximum(
        jnp.dot(xb, wd_ref[...], preferred_element_type=jnp.float32)
        + bd_ref[...], 0.0)

    @pl.when(nb == n_blocks - 1)
    def _finish():
        consume((n_blocks - 1) * _BLK, 1.0)
        afeat = acc_ref[...] / l_ref[...]
        o0 = jnp.sum(afeat * wc0_ref[...], axis=1, keepdims=True)
        o1 = jnp.sum(afeat * wc1_ref[...], axis=1, keepdims=True)
        outs_ref[...] = jnp.concatenate([o0, o1], axis=1) + bcr_ref[...]
        bag = jnp.dot(s_ref[...], afeat,
                      preferred_element_type=jnp.float32)
        b0 = jnp.sum(bag * ws0_ref[...], axis=1, keepdims=True)
        b1 = jnp.sum(bag * ws1_ref[...], axis=1, keepdims=True)
        bags_ref[...] = jnp.concatenate([b0, b1], axis=1) + bs_ref[...]


@jax.jit
def kernel(x, Wd, bd, Wv, bv, Wu, bu, Wa, ba, Wc, bc, Ws, bs):
    n = x.shape[1]
    d_feat = x.shape[2]
    d_inner = Wd.shape[1]
    n_task, _, d_att = Wv.shape
    n_token = Wa.shape[2]
    n_class = Wc.shape[3]
    r = n_task * n_token
    gate_cols = n_task * d_att
    n_blocks = n // _BLK

    x2 = x[0]
    Wvu = jnp.concatenate(
        [Wv.transpose(1, 0, 2).reshape(d_inner, gate_cols),
         Wu.transpose(1, 0, 2).reshape(d_inner, gate_cols)], axis=1)
    bvu = jnp.concatenate([bv.reshape(-1), bu.reshape(-1)])[None, :]
    eye_t = jnp.eye(n_task, dtype=Wa.dtype)
    WaT = jnp.einsum('ikj,im->mkij', Wa, eye_t).reshape(gate_cols, r)
    ba_col = ba.reshape(r, 1)
    Wc_r = Wc.reshape(r, d_inner, n_class)
    bc_r = bc.reshape(r, n_class)
    S = jnp.repeat(jnp.eye(n_task, dtype=x.dtype), n_token, axis=1) / n_token

    body = functools.partial(_fused_kernel, n_blocks, gate_cols)
    aout, outs, bags = pl.pallas_call(
        body,
        grid=(n_blocks,),
        in_specs=[
            pl.BlockSpec((_BLK, d_feat), lambda nb: (nb, 0)),
            pl.BlockSpec((d_feat, d_inner), lambda nb: (0, 0)),
            pl.BlockSpec((1, d_inner), lambda nb: (0, 0)),
            pl.BlockSpec((d_inner, 2 * gate_cols), lambda nb: (0, 0)),
            pl.BlockSpec((1, 2 * gate_cols), lambda nb: (0, 0)),
            pl.BlockSpec((gate_cols, r), lambda nb: (0, 0)),
            pl.BlockSpec((r, 1), lambda nb: (0, 0)),
            pl.BlockSpec((r, d_inner), lambda nb: (0, 0)),
            pl.BlockSpec((r, d_inner), lambda nb: (0, 0)),
            pl.BlockSpec((r, n_class), lambda nb: (0, 0)),
            pl.BlockSpec((n_task, d_inner), lambda nb: (0, 0)),
            pl.BlockSpec((n_task, d_inner), lambda nb: (0, 0)),
            pl.BlockSpec((n_task, n_class), lambda nb: (0, 0)),
            pl.BlockSpec((n_task, r), lambda nb: (0, 0)),
        ],
        out_specs=[
            pl.BlockSpec((r, n), lambda nb: (0, 0)),
            pl.BlockSpec((r, n_class), lambda nb: (0, 0)),
            pl.BlockSpec((n_task, n_class), lambda nb: (0, 0)),
        ],
        out_shape=[
            jax.ShapeDtypeStruct((r, n), jnp.float32),
            jax.ShapeDtypeStruct((r, n_class), jnp.float32),
            jax.ShapeDtypeStruct((n_task, n_class), jnp.float32),
        ],
        scratch_shapes=[
            pltpu.VMEM((_BLK, d_inner), jnp.float32),
            pltpu.VMEM((r, d_inner), jnp.float32),
            pltpu.VMEM((r, 1), jnp.float32),
        ],
    )(x2, Wd, bd[None, :], Wvu, bvu, WaT, ba_col,
      Wc_r[:, :, 0], Wc_r[:, :, 1], bc_r, Ws[:, :, 0], Ws[:, :, 1], bs, S)

    outs_full = outs.reshape(n_task, n_token, n_class)
    bags_full = bags.reshape(n_task, 1, n_class)
    aouts_full = aout.reshape(n_task, n_token, n)[:, None, :, :]
    return outs_full, bags_full, aouts_full


# speedup vs baseline: 1.4272x; 1.0718x over previous
import functools

import jax
import jax.numpy as jnp
from jax.experimental import pallas as pl
from jax.experimental.pallas import tpu as pltpu

_BLK = 1024


def _fused_kernel(n_blocks, gate_cols,
                  x_ref, wd_ref, bd_ref, wvu_ref, bvu_ref, wat_ref, ba_ref,
                  wc0_ref, wc1_ref, bcr_ref, ws0_ref, ws1_ref, bs_ref, s_ref,
                  aout_ref, outs_ref, bags_ref,
                  l_ref, acc_ref):
    nb = pl.program_id(0)

    @pl.when(nb == 0)
    def _init():
        l_ref[...] = jnp.zeros_like(l_ref)
        acc_ref[...] = jnp.zeros_like(acc_ref)

    xb = x_ref[...]
    h = jnp.maximum(
        jnp.dot(xb, wd_ref[...], preferred_element_type=jnp.float32)
        + bd_ref[...], 0.0)
    g = jnp.dot(h, wvu_ref[...], preferred_element_type=jnp.float32) \
        + bvu_ref[...]
    gate = jnp.tanh(g[:, :gate_cols]) * jax.nn.sigmoid(g[:, gate_cols:])
    a_t = jax.lax.dot_general(
        wat_ref[...], gate, (((0,), (1,)), ((), ())),
        preferred_element_type=jnp.float32) + ba_ref[...]
    aout_ref[...] = a_t

    p = jnp.exp(a_t)
    l_ref[...] = l_ref[...] + jnp.sum(p, axis=1, keepdims=True)
    acc_ref[...] = acc_ref[...] + jnp.dot(
        p, h, preferred_element_type=jnp.float32)

    @pl.when(nb == n_blocks - 1)
    def _finish():
        afeat = acc_ref[...] / l_ref[...]
        o0 = jnp.sum(afeat * wc0_ref[...], axis=1, keepdims=True)
        o1 = jnp.sum(afeat * wc1_ref[...], axis=1, keepdims=True)
        outs_ref[...] = jnp.concatenate([o0, o1], axis=1) + bcr_ref[...]
        bag = jnp.dot(s_ref[...], afeat,
                      preferred_element_type=jnp.float32)
        b0 = jnp.sum(bag * ws0_ref[...], axis=1, keepdims=True)
        b1 = jnp.sum(bag * ws1_ref[...], axis=1, keepdims=True)
        bags_ref[...] = jnp.concatenate([b0, b1], axis=1) + bs_ref[...]


@jax.jit
def kernel(x, Wd, bd, Wv, bv, Wu, bu, Wa, ba, Wc, bc, Ws, bs):
    n = x.shape[1]
    d_feat = x.shape[2]
    d_inner = Wd.shape[1]
    n_task, _, d_att = Wv.shape
    n_token = Wa.shape[2]
    n_class = Wc.shape[3]
    r = n_task * n_token
    gate_cols = n_task * d_att
    n_blocks = n // _BLK

    x2 = x[0]
    Wvu = jnp.concatenate(
        [Wv.transpose(1, 0, 2).reshape(d_inner, gate_cols),
         Wu.transpose(1, 0, 2).reshape(d_inner, gate_cols)], axis=1)
    bvu = jnp.concatenate([bv.reshape(-1), bu.reshape(-1)])[None, :]
    eye_t = jnp.eye(n_task, dtype=Wa.dtype)
    WaT = jnp.einsum('ikj,im->mkij', Wa, eye_t).reshape(gate_cols, r)
    ba_col = ba.reshape(r, 1)
    Wc_r = Wc.reshape(r, d_inner, n_class)
    bc_r = bc.reshape(r, n_class)
    S = jnp.repeat(jnp.eye(n_task, dtype=x.dtype), n_token, axis=1) / n_token

    body = functools.partial(_fused_kernel, n_blocks, gate_cols)
    aout, outs, bags = pl.pallas_call(
        body,
        grid=(n_blocks,),
        in_specs=[
            pl.BlockSpec((_BLK, d_feat), lambda nb: (nb, 0)),
            pl.BlockSpec((d_feat, d_inner), lambda nb: (0, 0)),
            pl.BlockSpec((1, d_inner), lambda nb: (0, 0)),
            pl.BlockSpec((d_inner, 2 * gate_cols), lambda nb: (0, 0)),
            pl.BlockSpec((1, 2 * gate_cols), lambda nb: (0, 0)),
            pl.BlockSpec((gate_cols, r), lambda nb: (0, 0)),
            pl.BlockSpec((r, 1), lambda nb: (0, 0)),
            pl.BlockSpec((r, d_inner), lambda nb: (0, 0)),
            pl.BlockSpec((r, d_inner), lambda nb: (0, 0)),
            pl.BlockSpec((r, n_class), lambda nb: (0, 0)),
            pl.BlockSpec((n_task, d_inner), lambda nb: (0, 0)),
            pl.BlockSpec((n_task, d_inner), lambda nb: (0, 0)),
            pl.BlockSpec((n_task, n_class), lambda nb: (0, 0)),
            pl.BlockSpec((n_task, r), lambda nb: (0, 0)),
        ],
        out_specs=[
            pl.BlockSpec((r, _BLK), lambda nb: (0, nb)),
            pl.BlockSpec((r, n_class), lambda nb: (0, 0)),
            pl.BlockSpec((n_task, n_class), lambda nb: (0, 0)),
        ],
        out_shape=[
            jax.ShapeDtypeStruct((r, n), jnp.float32),
            jax.ShapeDtypeStruct((r, n_class), jnp.float32),
            jax.ShapeDtypeStruct((n_task, n_class), jnp.float32),
        ],
        scratch_shapes=[
            pltpu.VMEM((r, 1), jnp.float32),
            pltpu.VMEM((r, d_inner), jnp.float32),
        ],
    )(x2, Wd, bd[None, :], Wvu, bvu, WaT, ba_col,
      Wc_r[:, :, 0], Wc_r[:, :, 1], bc_r, Ws[:, :, 0], Ws[:, :, 1], bs, S)

    outs_full = outs.reshape(n_task, n_token, n_class)
    bags_full = bags.reshape(n_task, 1, n_class)
    aouts_full = aout.reshape(n_task, n_token, n)[:, None, :, :]
    return outs_full, bags_full, aouts_full
